# baseline (speedup 1.0000x reference)
import functools

import jax
import jax.numpy as jnp
from jax import lax
from jax.experimental import pallas as pl
from jax.experimental.pallas import tpu as pltpu
from jax.experimental.pallas import tpu_sc as plsc

_B = 16384
_D = 64
_NW = 32
_BPW = _B // _NW
_CH = 128
_NCH = _BPW // _CH


def _sc_gather(user, item, tug, tig, tum, tim):
    mesh = plsc.VectorSubcoreMesh(core_axis_name="c", subcore_axis_name="s")
    out_t = [jax.ShapeDtypeStruct((_B, 2 * _D), jnp.float32) for _ in range(2)]
    scratch = (
        [pltpu.VMEM((_BPW,), jnp.int32) for _ in range(2)]
        + [pltpu.VMEM((_NCH, _CH), jnp.int32) for _ in range(2)]
        + [pltpu.VMEM((_CH, 2 * _D), jnp.float32) for _ in range(6)]
        + [pltpu.SemaphoreType.DMA for _ in range(4)]
    )

    @functools.partial(pl.kernel, mesh=mesh, out_type=out_t,
                       scratch_types=scratch)
    def body(user_h, item_h, tug_h, tig_h, tum_h, tim_h, o_gmf, o_mlp,
             idxu, idxi, pidxu, pidxi, ub0, ub1, ib0, ib1, ob0, ob1,
             gsem0, gsem1, wsem0, wsem1):
        c = lax.axis_index("c")
        s = lax.axis_index("s")
        base = (s * 2 + c) * _BPW

        pltpu.sync_copy(user_h.at[pl.ds(base, _BPW)], idxu)
        pltpu.sync_copy(item_h.at[pl.ds(base, _BPW)], idxi)

        for j in range(_NCH):
            for g in range(_CH // 16):
                pidxu[j, pl.ds(g * 16, 16)] = lax.shift_right_logical(
                    idxu[pl.ds(j * _CH + g * 16, 16)], 1)
                pidxi[j, pl.ds(g * 16, 16)] = lax.shift_right_logical(
                    idxi[pl.ds(j * _CH + g * 16, 16)], 1)

        ubufs, ibufs, obufs = [ub0, ub1], [ib0, ib1], [ob0, ob1]
        gsems, wsems = [gsem0, gsem1], [wsem0, wsem1]

        phases = [(tug_h, tig_h, o_gmf, j) for j in range(_NCH)] \
               + [(tum_h, tim_h, o_mlp, j) for j in range(_NCH)]

        def fire(p):
            tu, ti, _, j = phases[p]
            k = p % 2
            return [
                pltpu.async_copy(tu.at[pidxu.at[j]], ubufs[k], gsems[k]),
                pltpu.async_copy(ti.at[pidxi.at[j]], ibufs[k], gsems[k]),
            ]

        def compact(p):
            _, _, _, j = phases[p]
            k = p % 2
            ub, ib, ob = ubufs[k], ibufs[k], obufs[k]

            def cbody(g, carry):
                vu = idxu[pl.ds(j * _CH + g * 16, 16)]
                vi = idxi[pl.ds(j * _CH + g * 16, 16)]
                for l in range(16):
                    r = g * 16 + l
                    hu = (vu[l] & 1) * _D
                    hi = (vi[l] & 1) * _D
                    for q in range(_D // 16):
                        ob[r, pl.ds(q * 16, 16)] = \
                            ub[r, pl.ds(hu + q * 16, 16)]
                        ob[r, pl.ds(_D + q * 16, 16)] = \
                            ib[r, pl.ds(hi + q * 16, 16)]
                return carry
            lax.fori_loop(0, _CH // 16, cbody, 0)

        inflight = fire(0)
        write_handles = [None, None]
        for p in range(8):
            for h in inflight:
                h.wait()
            if p < 7:
                nxt = fire(p + 1)
            else:
                nxt = []
            k = p % 2
            if write_handles[k] is not None:
                write_handles[k].wait()
            compact(p)
            _, _, out, j = phases[p]
            write_handles[k] = pltpu.async_copy(
                obufs[k], out.at[pl.ds(base + j * _CH, _CH)], wsems[k])
            inflight = nxt
        for wh in write_handles:
            wh.wait()

    v2 = tug.shape[0] // 2
    return body(user, item,
                tug.reshape(v2, 2 * _D), tig.reshape(v2, 2 * _D),
                tum.reshape(v2, 2 * _D), tim.reshape(v2, 2 * _D))


def _tc_mlp(gmf2, mlp2, W1, b1, W2, b2, W3, b3, Wout, bout):
    bs = 2048
    grid = (_B // bs,)
    b1r = b1.reshape(1, -1)
    b2r = b2.reshape(1, -1)
    b3r = b3.reshape(1, -1)
    wa = Wout[:_D, 0].reshape(1, _D)
    wb = Wout[_D:, 0].reshape(1, -1)
    bor = bout.reshape(1, 1)

    def body(g_r, m_r, w1_r, b1_r, w2_r, b2_r, w3_r, b3_r,
             wa_r, wb_r, bo_r, out_r):
        h = jnp.dot(m_r[...], w1_r[...], preferred_element_type=jnp.float32)
        h = jax.nn.relu(h + b1_r[...])
        h = jax.nn.relu(jnp.dot(h, w2_r[...],
                                preferred_element_type=jnp.float32) + b2_r[...])
        h = jax.nn.relu(jnp.dot(h, w3_r[...],
                                preferred_element_type=jnp.float32) + b3_r[...])
        g = g_r[:, :_D] * g_r[:, _D:]
        p = (jnp.sum(g * wa_r[...], axis=1, keepdims=True)
             + jnp.sum(h * wb_r[...], axis=1, keepdims=True) + bo_r[0, 0])
        out_r[...] = 1.0 / (1.0 + jnp.exp(-p))

    full = lambda a: pl.BlockSpec(a.shape, lambda i: (0,) * a.ndim)
    emb = pl.BlockSpec((bs, 2 * _D), lambda i: (i, 0))
    out = pl.pallas_call(
        body,
        grid=grid,
        in_specs=[emb, emb,
                  full(W1), full(b1r), full(W2), full(b2r),
                  full(W3), full(b3r), full(wa), full(wb), full(bor)],
        out_specs=pl.BlockSpec((bs, 1), lambda i: (i, 0)),
        out_shape=jax.ShapeDtypeStruct((_B, 1), jnp.float32),
    )(gmf2, mlp2, W1, b1r, W2, b2r, W3, b3r, wa, wb, bor)
    return out.reshape(_B)


def kernel(user, item, user_gmf_emb, item_gmf_emb, user_mlp_emb, item_mlp_emb,
           W1, b1, W2, b2, W3, b3, Wout, bout):
    gmf2, mlp2 = _sc_gather(user, item, user_gmf_emb, item_gmf_emb,
                            user_mlp_emb, item_mlp_emb)
    return _tc_mlp(gmf2, mlp2, W1, b1, W2, b2, W3, b3, Wout, bout)
